# baseline (speedup 1.0000x reference)
import functools

import jax
import jax.numpy as jnp
from jax import lax
from jax.experimental import pallas as pl
from jax.experimental.pallas import tpu as pltpu
from jax.experimental.pallas import tpu_sc as plsc

_B = 320000
_D = 128
_L = 16
_NC = 2
_NS = 16
_NW = _NC * _NS
_PER_W = _B // _NW
_C = 80
_NCHUNK = _PER_W // _C


def _tec_body(z_hbm, e0_hbm, e1_hbm, out_hbm,
              idx0_f, idx1_f, rows0_a, rows1_a, rows0_b, rows1_b, res_f,
              sa0, sa1, sb0, sb1):
    wid = lax.axis_index("s") * _NC + lax.axis_index("c")
    base = wid * _PER_W

    pltpu.sync_copy(e0_hbm.at[pl.ds(base, _PER_W)], idx0_f)
    pltpu.sync_copy(e1_hbm.at[pl.ds(base, _PER_W)], idx1_f)

    lanes = lax.iota(jnp.int32, _L)

    def issue(i, r0, r1, s0, s1):
        off = i * _C
        pltpu.async_copy(z_hbm.at[idx0_f.at[pl.ds(off, _C)]], r0, s0)
        pltpu.async_copy(z_hbm.at[idx1_f.at[pl.ds(off, _C)]], r1, s1)

    def wait(i, r0, r1, s0, s1):
        off = i * _C
        pltpu.make_async_copy(z_hbm.at[idx0_f.at[pl.ds(off, _C)]], r0, s0).wait()
        pltpu.make_async_copy(z_hbm.at[idx1_f.at[pl.ds(off, _C)]], r1, s1).wait()

    def compute(i, r0, r1):
        nacc = 8

        def group(g, _):
            row = g * _L + lanes
            idx = [jnp.full((_L,), k, jnp.int32) for k in range(nacc)]
            acc = [plsc.load_gather(r0, [row, idx[k]])
                   * plsc.load_gather(r1, [row, idx[k]]) for k in range(nacc)]
            for _step in range(1, _D // nacc):
                for k in range(nacc):
                    idx[k] = idx[k] + nacc
                    acc[k] = acc[k] + (plsc.load_gather(r0, [row, idx[k]])
                                       * plsc.load_gather(r1, [row, idx[k]]))
            tot = ((acc[0] + acc[1]) + (acc[2] + acc[3])) + (
                (acc[4] + acc[5]) + (acc[6] + acc[7]))
            res_f[pl.ds(i * _C + g * _L, _L)] = 1.0 / (1.0 + jnp.exp(-tot))
            return ()

        lax.fori_loop(0, _C // _L, group, ())

    issue(0, rows0_a, rows1_a, sa0, sa1)
    issue(1, rows0_b, rows1_b, sb0, sb1)

    def body(j, _):
        i0 = 2 * j
        i1 = 2 * j + 1
        wait(i0, rows0_a, rows1_a, sa0, sa1)
        compute(i0, rows0_a, rows1_a)
        issue(i0 + 2, rows0_a, rows1_a, sa0, sa1)
        wait(i1, rows0_b, rows1_b, sb0, sb1)
        compute(i1, rows0_b, rows1_b)

        @pl.when(i1 + 2 < _NCHUNK)
        def _():
            issue(i1 + 2, rows0_b, rows1_b, sb0, sb1)

        return ()

    lax.fori_loop(0, _NCHUNK // 2, body, ())

    wait(_NCHUNK - 1, rows0_a, rows1_a, sa0, sa1)
    compute(_NCHUNK - 1, rows0_a, rows1_a)
    pltpu.sync_copy(res_f, out_hbm.at[pl.ds(base, _PER_W)])


@functools.partial(jax.jit, static_argnums=())
def _sc_call(z, e0, e1):
    mesh = plsc.VectorSubcoreMesh(core_axis_name="c", subcore_axis_name="s")
    f = pl.kernel(
        _tec_body,
        mesh=mesh,
        compiler_params=pltpu.CompilerParams(needs_layout_passes=False),
        out_type=jax.ShapeDtypeStruct((_B,), jnp.float32),
        scratch_types=[
            pltpu.VMEM((_PER_W,), jnp.int32),
            pltpu.VMEM((_PER_W,), jnp.int32),
            pltpu.VMEM((_C, _D), jnp.float32),
            pltpu.VMEM((_C, _D), jnp.float32),
            pltpu.VMEM((_C, _D), jnp.float32),
            pltpu.VMEM((_C, _D), jnp.float32),
            pltpu.VMEM((_PER_W,), jnp.float32),
            pltpu.SemaphoreType.DMA,
            pltpu.SemaphoreType.DMA,
            pltpu.SemaphoreType.DMA,
            pltpu.SemaphoreType.DMA,
        ],
    )
    return f(z, e0, e1)


def kernel(z, e):
    e0 = e[0].astype(jnp.int32)
    e1 = e[1].astype(jnp.int32)
    return _sc_call(z, e0, e1)


# speedup vs baseline: 6.4177x; 6.4177x over previous
import functools

import jax
import jax.numpy as jnp
from jax import lax
from jax.experimental import pallas as pl
from jax.experimental.pallas import tpu as pltpu
from jax.experimental.pallas import tpu_sc as plsc

_B = 320000
_D = 128
_L = 16
_NC = 2
_NS = 16
_NW = _NC * _NS
_PER_W = _B // _NW
_C = 80
_NCHUNK = _PER_W // _C


def _tec_body(z_hbm, e0_hbm, e1_hbm, out_hbm,
              idx0_f, idx1_f, rows0_a, rows1_a, rows0_b, rows1_b, res_f,
              part_v, sa0, sa1, sb0, sb1):
    wid = lax.axis_index("s") * _NC + lax.axis_index("c")
    base = wid * _PER_W

    pltpu.sync_copy(e0_hbm.at[pl.ds(base, _PER_W)], idx0_f)
    pltpu.sync_copy(e1_hbm.at[pl.ds(base, _PER_W)], idx1_f)

    lanes = lax.iota(jnp.int32, _L)

    def issue(i, r0, r1, s0, s1):
        off = i * _C
        pltpu.async_copy(z_hbm.at[idx0_f.at[pl.ds(off, _C)]], r0, s0)
        pltpu.async_copy(z_hbm.at[idx1_f.at[pl.ds(off, _C)]], r1, s1)

    def wait(i, r0, r1, s0, s1):
        off = i * _C
        pltpu.make_async_copy(z_hbm.at[idx0_f.at[pl.ds(off, _C)]], r0, s0).wait()
        pltpu.make_async_copy(z_hbm.at[idx1_f.at[pl.ds(off, _C)]], r1, s1).wait()

    lanes17 = lanes * 17

    def compute(i, r0, r1):
        def group(g, _):
            base_c = g * _L
            for e2 in range(_L):
                c = base_c + e2
                s = [r0[c, pl.ds(l * _L, _L)] * r1[c, pl.ds(l * _L, _L)]
                     for l in range(_D // _L)]
                acc = (((s[0] + s[1]) + (s[2] + s[3]))
                       + ((s[4] + s[5]) + (s[6] + s[7])))
                part_v[pl.ds(e2 * 17, _L)] = acc
            t = [plsc.load_gather(part_v, [lanes17 + l]) for l in range(_L)]
            t = [t[2 * k] + t[2 * k + 1] for k in range(8)]
            t = [t[2 * k] + t[2 * k + 1] for k in range(4)]
            tot = (t[0] + t[1]) + (t[2] + t[3])
            res_f[pl.ds(i * _C + base_c, _L)] = 1.0 / (1.0 + jnp.exp(-tot))
            return ()

        lax.fori_loop(0, _C // _L, group, ())

    issue(0, rows0_a, rows1_a, sa0, sa1)
    issue(1, rows0_b, rows1_b, sb0, sb1)

    def body(j, _):
        i0 = 2 * j
        i1 = 2 * j + 1
        wait(i0, rows0_a, rows1_a, sa0, sa1)
        compute(i0, rows0_a, rows1_a)
        issue(i0 + 2, rows0_a, rows1_a, sa0, sa1)
        wait(i1, rows0_b, rows1_b, sb0, sb1)
        compute(i1, rows0_b, rows1_b)

        @pl.when(i1 + 2 < _NCHUNK)
        def _():
            issue(i1 + 2, rows0_b, rows1_b, sb0, sb1)

        return ()

    lax.fori_loop(0, _NCHUNK // 2, body, ())

    wait(_NCHUNK - 1, rows0_a, rows1_a, sa0, sa1)
    compute(_NCHUNK - 1, rows0_a, rows1_a)
    pltpu.sync_copy(res_f, out_hbm.at[pl.ds(base, _PER_W)])


@functools.partial(jax.jit, static_argnums=())
def _sc_call(z, e0, e1):
    mesh = plsc.VectorSubcoreMesh(core_axis_name="c", subcore_axis_name="s")
    f = pl.kernel(
        _tec_body,
        mesh=mesh,
        compiler_params=pltpu.CompilerParams(needs_layout_passes=False),
        out_type=jax.ShapeDtypeStruct((_B,), jnp.float32),
        scratch_types=[
            pltpu.VMEM((_PER_W,), jnp.int32),
            pltpu.VMEM((_PER_W,), jnp.int32),
            pltpu.VMEM((_C, _D), jnp.float32),
            pltpu.VMEM((_C, _D), jnp.float32),
            pltpu.VMEM((_C, _D), jnp.float32),
            pltpu.VMEM((_C, _D), jnp.float32),
            pltpu.VMEM((_PER_W,), jnp.float32),
            pltpu.VMEM((_L * 17,), jnp.float32),
            pltpu.SemaphoreType.DMA,
            pltpu.SemaphoreType.DMA,
            pltpu.SemaphoreType.DMA,
            pltpu.SemaphoreType.DMA,
        ],
    )
    return f(z, e0, e1)


def kernel(z, e):
    e0 = e[0].astype(jnp.int32)
    e1 = e[1].astype(jnp.int32)
    return _sc_call(z, e0, e1)
